# baseline (speedup 1.0000x reference)
import functools

import jax
import jax.numpy as jnp
from jax import lax
from jax.experimental import pallas as pl
from jax.experimental.pallas import tpu as pltpu
from jax.experimental.pallas import tpu_sc as plsc

_N = 10000
_E = 320000
_D = 128
_HD = _D // 2
_NC = 2
_NS = 16
_C = 80
_EPS = _E // _NS
_CH = _EPS // _C
_NB = 5
_RPT = 632
_RPT_LAST = _N - 15 * _RPT
_RB = 1000


@functools.partial(
    pl.kernel,
    out_type=jax.ShapeDtypeStruct((_NC, _N, _HD), jnp.float32),
    mesh=plsc.VectorSubcoreMesh(core_axis_name="c", subcore_axis_name="s"),
    compiler_params=pltpu.CompilerParams(use_tc_tiling_on_sc=False,
                                        needs_layout_passes=False),
    scratch_types=[
        pltpu.VMEM((_CH, _C), jnp.int32),
        pltpu.VMEM((2 * _NB, _C), jnp.int32),
        pltpu.VMEM((2 * _NB, _C), jnp.int32),
        [pltpu.VMEM((_C, _HD), jnp.float32) for _ in range(_NB)],
        [pltpu.VMEM((_C, _HD), jnp.float32) for _ in range(_NB)],
        pltpu.VMEM_SHARED((_N + 8, _HD), jnp.float32),
        [pltpu.SemaphoreType.DMA for _ in range(_NB)],
        [pltpu.SemaphoreType.DMA for _ in range(_NB)],
    ],
)
def _sc_agg(xe_hbm, xo_hbm, pck_hbm, z_hbm, out_hbm, pck_v, srcr,
            idxr, rows, stg, acc, gsem, ssem):
    c = lax.axis_index("c")
    s = lax.axis_index("s")

    def fetch(j, slot):
        for m in range(_C // 16):
            v = pck_v[j, pl.ds(16 * m, 16)]
            srcr[slot, pl.ds(16 * m, 16)] = v & 0x3FFF
            idxr[slot, pl.ds(16 * m, 16)] = (v >> 14) & 0x3FFF

        @pl.when(c == 0)
        def _():
            pltpu.async_copy(xe_hbm.at[srcr.at[slot]], rows[slot % _NB],
                             gsem[slot % _NB])

        @pl.when(c == 1)
        def _():
            pltpu.async_copy(xo_hbm.at[srcr.at[slot]], rows[slot % _NB],
                             gsem[slot % _NB])

    pltpu.sync_copy(z_hbm, acc.at[pl.ds(s * _RPT, 320)])

    @pl.when(s < _NS - 1)
    def _():
        pltpu.sync_copy(z_hbm.at[pl.ds(0, _RPT - 320)],
                        acc.at[pl.ds(s * _RPT + 320, _RPT - 320)])

    @pl.when(s == _NS - 1)
    def _():
        pltpu.sync_copy(z_hbm.at[pl.ds(0, _RPT_LAST - 320)],
                        acc.at[pl.ds(15 * _RPT + 320, _RPT_LAST - 320)])

    pltpu.sync_copy(pck_hbm.at[s], pck_v)
    plsc.subcore_barrier()

    for k in range(_NB - 1):
        fetch(k, k)

    zed = jnp.zeros((16,), jnp.float32)

    def step(i, car):
        for k in range(2 * _NB):
            j = 2 * _NB * i + k
            b = k % _NB
            pltpu.make_async_copy(xe_hbm.at[srcr.at[0]], rows[b],
                                  gsem[b]).wait()

            @pl.when(j >= _NB)
            def _():
                pltpu.make_async_copy(stg[b], acc.at[idxr.at[0]],
                                      ssem[b]).wait()

            def one_row(r, cs, b=b, j=j):
                v = plsc.load_gather(
                    pck_v, [jnp.full((16,), j, jnp.int32),
                            jnp.full((16,), r, jnp.int32)])
                sm = (-(v >> 31)).astype(jnp.float32)
                out = []
                for q, cq in enumerate(cs):
                    rq = rows[b][r, pl.ds(16 * q, 16)]
                    nq = cq * sm + rq
                    stg[b][r, pl.ds(16 * q, 16)] = nq
                    out.append(nq)
                return tuple(out)

            def fold_row(r2, fc):
                return one_row(2 * r2 + 1, one_row(2 * r2, fc))

            car = lax.fori_loop(0, _C // 2, fold_row, car)
            pltpu.async_copy(stg[b], acc.at[idxr.at[k]], ssem[b],
                             add=True)
            j4 = j + _NB - 1
            k4 = (k + _NB - 1) % (2 * _NB)

            @pl.when(j4 < _CH)
            def _():
                fetch(j4, k4)
        return car

    lax.fori_loop(0, _CH // (2 * _NB), step, (zed, zed, zed, zed))
    for k in range(_NB):
        pltpu.make_async_copy(stg[k], acc.at[idxr.at[0]], ssem[k]).wait()

    plsc.subcore_barrier()

    @pl.when(s < _NS - 1)
    def _():
        pltpu.sync_copy(acc.at[pl.ds(s * _RPT, _RPT)],
                        out_hbm.at[c, pl.ds(s * _RPT, _RPT)])

    @pl.when(s == _NS - 1)
    def _():
        pltpu.sync_copy(acc.at[pl.ds(15 * _RPT, _RPT_LAST)],
                        out_hbm.at[c, pl.ds(15 * _RPT, _RPT_LAST)])


def _bdot(a, b):
    return jnp.dot(a.astype(jnp.bfloat16), b.astype(jnp.bfloat16),
                   preferred_element_type=jnp.float32)


def _mlp(xe, xo, agg, W1, b1, W2, b2):
    def body(xe_ref, xo_ref, agg_ref, w1_ref, b1_ref, w2_ref, b2_ref,
             oe_ref, oo_ref):
        h = (jnp.concatenate([xe_ref[...], xo_ref[...]], axis=1)
             + jnp.concatenate([agg_ref[0], agg_ref[1]], axis=1))
        h1 = jnp.maximum(_bdot(h, w1_ref[...]) + b1_ref[...], 0.0)
        o = jnp.maximum(_bdot(h1, w2_ref[...]) + b2_ref[...], 0.0)
        oe_ref[...] = o[:, :_HD]
        oo_ref[...] = o[:, _HD:]

    return pl.pallas_call(
        body,
        grid=(_N // _RB,),
        in_specs=[
            pl.BlockSpec((_RB, _HD), lambda i: (i, 0)),
            pl.BlockSpec((_RB, _HD), lambda i: (i, 0)),
            pl.BlockSpec((_NC, _RB, _HD), lambda i: (0, i, 0)),
            pl.BlockSpec((_D, _D), lambda i: (0, 0)),
            pl.BlockSpec((1, _D), lambda i: (0, 0)),
            pl.BlockSpec((_D, _D), lambda i: (0, 0)),
            pl.BlockSpec((1, _D), lambda i: (0, 0)),
        ],
        out_specs=[
            pl.BlockSpec((_RB, _HD), lambda i: (i, 0)),
            pl.BlockSpec((_RB, _HD), lambda i: (i, 0)),
        ],
        out_shape=[
            jax.ShapeDtypeStruct((_N, _HD), jnp.float32),
            jax.ShapeDtypeStruct((_N, _HD), jnp.float32),
        ],
    )(xe, xo, agg, W1, b1.reshape(1, _D), W2, b2.reshape(1, _D))


def _attn(x3, Wa, ba, Wb, bb, Wc, bc, Wn, bn):
    def body(x_ref, wa, ba_, wb, bb_, wc, bc_, wn, bn_, a_ref, yn_ref):
        xb = x_ref[...]
        a = jnp.tanh(_bdot(xb, wa[...]) + ba_[...])
        b = jax.nn.sigmoid(_bdot(xb, wb[...]) + bb_[...])
        a_ref[...] = _bdot(a * b, wc[...]) + bc_[...]
        yn_ref[...] = _bdot(xb, wn[...]) + bn_[...]

    nn = bn.shape[0]
    return pl.pallas_call(
        body,
        grid=(_N // _RB,),
        in_specs=[
            pl.BlockSpec((_RB, _D), lambda i: (i, 0)),
            pl.BlockSpec((_D, _D), lambda i: (0, 0)),
            pl.BlockSpec((1, _D), lambda i: (0, 0)),
            pl.BlockSpec((_D, _D), lambda i: (0, 0)),
            pl.BlockSpec((1, _D), lambda i: (0, 0)),
            pl.BlockSpec((_D, 1), lambda i: (0, 0)),
            pl.BlockSpec((1, 1), lambda i: (0, 0)),
            pl.BlockSpec((_D, nn), lambda i: (0, 0)),
            pl.BlockSpec((1, nn), lambda i: (0, 0)),
        ],
        out_specs=[
            pl.BlockSpec((_RB, 1), lambda i: (i, 0)),
            pl.BlockSpec((_RB, nn), lambda i: (i, 0)),
        ],
        out_shape=[
            jax.ShapeDtypeStruct((_N, 1), jnp.float32),
            jax.ShapeDtypeStruct((_N, nn), jnp.float32),
        ],
    )(x3, Wa, ba.reshape(1, _D), Wb, bb.reshape(1, _D), Wc,
      bc.reshape(1, 1), Wn, bn.reshape(1, nn))


def _head(A_t, x3, Wr, br, Wcls, bcls):
    nc = bcls.shape[0]

    def body(a_ref, x_ref, wr, br_, wcls, bcls_, lg_ref, pr_ref, yh_ref,
             s_ref):
        A = a_ref[...]
        m = jnp.max(A, axis=1, keepdims=True)
        e = jnp.exp(A - m)
        p = e / jnp.sum(e, axis=1, keepdims=True)
        hp = _bdot(p, x_ref[...])
        hr = jnp.maximum(_bdot(hp, wr[...]) + br_[...], 0.0)
        lg = _bdot(hr, wcls[...]) + bcls_[...]
        lg_ref[...] = lg
        lm = jnp.max(lg, axis=1, keepdims=True)
        el = jnp.exp(lg - lm)
        pr_ref[...] = el / jnp.sum(el, axis=1, keepdims=True)
        col = lax.broadcasted_iota(jnp.int32, (1, nc), 1)
        yh_ref[...] = jnp.min(jnp.where(lg == lm, col, nc), axis=1,
                              keepdims=True)
        haz = jax.nn.sigmoid(lg)
        lgp = jnp.log(1.0 - haz)
        parts = [lgp[:, 0:1]]
        for k in range(1, nc):
            parts.append(parts[-1] + lgp[:, k:k + 1])
        s_ref[...] = jnp.exp(jnp.concatenate(parts, axis=1))

    return pl.pallas_call(
        body,
        out_shape=[
            jax.ShapeDtypeStruct((1, nc), jnp.float32),
            jax.ShapeDtypeStruct((1, nc), jnp.float32),
            jax.ShapeDtypeStruct((1, 1), jnp.int32),
            jax.ShapeDtypeStruct((1, nc), jnp.float32),
        ],
    )(A_t, x3, Wr, br.reshape(1, _D), Wcls, bcls.reshape(1, nc))


def kernel(x, edge_index, batch, W11, b11, W12, b12, W21, b21, W22, b22,
           W31, b31, W32, b32, Wa, ba, Wb, bb, Wc, bc, Wr, br, Wcls, bcls,
           Wn, bn):
    perm = jnp.argsort(edge_index[1], stable=True)
    sp = edge_index[0][perm]
    dp = edge_index[1][perm]
    nxt = jnp.concatenate([dp[1:], jnp.full((1,), -1, jnp.int32)])
    last = (dp != nxt).reshape(_NS, _EPS)
    last = last.at[:, -1].set(True)
    dpr = dp.reshape(_NS, _EPS)
    idxa = jnp.where(last, dpr, _N)
    cont = jnp.concatenate(
        [jnp.zeros((_NS, 1), jnp.int32),
         (dpr[:, 1:] == dpr[:, :-1]).astype(jnp.int32)], axis=1)
    pck = ((cont << 31) | (idxa << 14)
           | sp.reshape(_NS, _EPS)).reshape(_NS, _CH, _C)
    z = jnp.zeros((320, _HD), jnp.float32)

    def agg(xe, xo):
        return _sc_agg(xe, xo, pck, z)

    xe0, xo0 = x[:, :_HD], x[:, _HD:]
    xe1, xo1 = _mlp(xe0, xo0, agg(xe0, xo0), W11, b11, W12, b12)
    xe2, xo2 = _mlp(xe1, xo1, agg(xe1, xo1), W21, b21, W22, b22)
    xe3, xo3 = _mlp(xe2, xo2, agg(xe2, xo2), W31, b31, W32, b32)
    x3 = jnp.concatenate([xe3, xo3], axis=1)
    A, Y_node = _attn(x3, Wa, ba, Wb, bb, Wc, bc, Wn, bn)
    A_raw = A.reshape(1, _N)
    logits, Y_prob, Y_hat, S = _head(A_raw, x3, Wr, br, Wcls, bcls)
    return (logits, Y_prob, Y_hat, A_raw, S, Y_node)


# speedup vs baseline: 1.0494x; 1.0494x over previous
import functools

import jax
import jax.numpy as jnp
from jax import lax
from jax.experimental import pallas as pl
from jax.experimental.pallas import tpu as pltpu
from jax.experimental.pallas import tpu_sc as plsc

_N = 10000
_E = 320000
_D = 128
_HD = _D // 2
_NC = 2
_NS = 16
_C = 80
_EPS = _E // _NS
_CH = _EPS // _C
_NB = 5
_RPT = 632
_RPT_LAST = _N - 15 * _RPT
_RB = 1000


@functools.partial(
    pl.kernel,
    out_type=jax.ShapeDtypeStruct((_NC, _N, _HD), jnp.float32),
    mesh=plsc.VectorSubcoreMesh(core_axis_name="c", subcore_axis_name="s"),
    compiler_params=pltpu.CompilerParams(use_tc_tiling_on_sc=False,
                                        needs_layout_passes=False),
    scratch_types=[
        pltpu.VMEM((_CH, _C), jnp.int32),
        pltpu.VMEM((2 * _NB, _C), jnp.int32),
        pltpu.VMEM((2 * _NB, _C), jnp.int32),
        [pltpu.VMEM((_C, _HD), jnp.float32) for _ in range(_NB)],
        [pltpu.VMEM((_C, _HD), jnp.float32) for _ in range(_NB)],
        pltpu.VMEM_SHARED((_N + 8, _HD), jnp.float32),
        [pltpu.SemaphoreType.DMA for _ in range(_NB)],
        [pltpu.SemaphoreType.DMA for _ in range(_NB)],
    ],
)
def _sc_agg(xe_hbm, xo_hbm, pck_hbm, z_hbm, out_hbm, pck_v, srcr,
            idxr, rows, stg, acc, gsem, ssem):
    c = lax.axis_index("c")
    s = lax.axis_index("s")

    def fetch(j, slot):
        for m in range(_C // 16):
            v = pck_v[j, pl.ds(16 * m, 16)]
            srcr[slot, pl.ds(16 * m, 16)] = v & 0x3FFF
            idxr[slot, pl.ds(16 * m, 16)] = v >> 14

        @pl.when(c == 0)
        def _():
            pltpu.async_copy(xe_hbm.at[srcr.at[slot]], rows[slot % _NB],
                             gsem[slot % _NB])

        @pl.when(c == 1)
        def _():
            pltpu.async_copy(xo_hbm.at[srcr.at[slot]], rows[slot % _NB],
                             gsem[slot % _NB])

    pltpu.sync_copy(z_hbm, acc.at[pl.ds(s * _RPT, 320)])

    @pl.when(s < _NS - 1)
    def _():
        pltpu.sync_copy(z_hbm.at[pl.ds(0, _RPT - 320)],
                        acc.at[pl.ds(s * _RPT + 320, _RPT - 320)])

    @pl.when(s == _NS - 1)
    def _():
        pltpu.sync_copy(z_hbm.at[pl.ds(0, _RPT_LAST - 320)],
                        acc.at[pl.ds(15 * _RPT + 320, _RPT_LAST - 320)])

    pltpu.sync_copy(pck_hbm.at[s], pck_v)
    plsc.subcore_barrier()

    for k in range(_NB - 1):
        fetch(k, k)

    zed = jnp.zeros((16,), jnp.float32)

    def step(i, car):
        for k in range(2 * _NB):
            j = 2 * _NB * i + k
            b = k % _NB
            pltpu.make_async_copy(xe_hbm.at[srcr.at[0]], rows[b],
                                  gsem[b]).wait()

            @pl.when(j >= _NB)
            def _():
                pltpu.make_async_copy(stg[b], acc.at[idxr.at[0]],
                                      ssem[b]).wait()

            def fold_row(r, fc, b=b, k=k):
                c0, c1, c2, c3, prev = fc
                d = plsc.load_gather(
                    idxr, [jnp.full((16,), k, jnp.int32),
                           jnp.full((16,), r, jnp.int32)])
                same = prev == _N
                out = []
                for q, cq in enumerate((c0, c1, c2, c3)):
                    rq = rows[b][r, pl.ds(16 * q, 16)]
                    nq = jnp.where(same, cq + rq, rq)
                    stg[b][r, pl.ds(16 * q, 16)] = nq
                    out.append(nq)
                return (out[0], out[1], out[2], out[3], d)

            car = lax.fori_loop(0, _C, fold_row, car)
            pltpu.async_copy(stg[b], acc.at[idxr.at[k]], ssem[b],
                             add=True)
            j4 = j + _NB - 1
            k4 = (k + _NB - 1) % (2 * _NB)

            @pl.when(j4 < _CH)
            def _():
                fetch(j4, k4)
        return car

    lax.fori_loop(0, _CH // (2 * _NB), step,
                  (zed, zed, zed, zed, jnp.full((16,), -1, jnp.int32)))
    for k in range(_NB):
        pltpu.make_async_copy(stg[k], acc.at[idxr.at[0]], ssem[k]).wait()

    plsc.subcore_barrier()

    @pl.when(s < _NS - 1)
    def _():
        pltpu.sync_copy(acc.at[pl.ds(s * _RPT, _RPT)],
                        out_hbm.at[c, pl.ds(s * _RPT, _RPT)])

    @pl.when(s == _NS - 1)
    def _():
        pltpu.sync_copy(acc.at[pl.ds(15 * _RPT, _RPT_LAST)],
                        out_hbm.at[c, pl.ds(15 * _RPT, _RPT_LAST)])


def _bdot(a, b):
    return jnp.dot(a.astype(jnp.bfloat16), b.astype(jnp.bfloat16),
                   preferred_element_type=jnp.float32)


def _mlp(xe, xo, agg, W1, b1, W2, b2):
    def body(xe_ref, xo_ref, agg_ref, w1_ref, b1_ref, w2_ref, b2_ref,
             oe_ref, oo_ref):
        h = (jnp.concatenate([xe_ref[...], xo_ref[...]], axis=1)
             + jnp.concatenate([agg_ref[0], agg_ref[1]], axis=1))
        h1 = jnp.maximum(_bdot(h, w1_ref[...]) + b1_ref[...], 0.0)
        o = jnp.maximum(_bdot(h1, w2_ref[...]) + b2_ref[...], 0.0)
        oe_ref[...] = o[:, :_HD]
        oo_ref[...] = o[:, _HD:]

    return pl.pallas_call(
        body,
        grid=(_N // _RB,),
        in_specs=[
            pl.BlockSpec((_RB, _HD), lambda i: (i, 0)),
            pl.BlockSpec((_RB, _HD), lambda i: (i, 0)),
            pl.BlockSpec((_NC, _RB, _HD), lambda i: (0, i, 0)),
            pl.BlockSpec((_D, _D), lambda i: (0, 0)),
            pl.BlockSpec((1, _D), lambda i: (0, 0)),
            pl.BlockSpec((_D, _D), lambda i: (0, 0)),
            pl.BlockSpec((1, _D), lambda i: (0, 0)),
        ],
        out_specs=[
            pl.BlockSpec((_RB, _HD), lambda i: (i, 0)),
            pl.BlockSpec((_RB, _HD), lambda i: (i, 0)),
        ],
        out_shape=[
            jax.ShapeDtypeStruct((_N, _HD), jnp.float32),
            jax.ShapeDtypeStruct((_N, _HD), jnp.float32),
        ],
    )(xe, xo, agg, W1, b1.reshape(1, _D), W2, b2.reshape(1, _D))


def _attn(x3, Wa, ba, Wb, bb, Wc, bc, Wn, bn):
    def body(x_ref, wa, ba_, wb, bb_, wc, bc_, wn, bn_, a_ref, yn_ref):
        xb = x_ref[...]
        a = jnp.tanh(_bdot(xb, wa[...]) + ba_[...])
        b = jax.nn.sigmoid(_bdot(xb, wb[...]) + bb_[...])
        a_ref[...] = _bdot(a * b, wc[...]) + bc_[...]
        yn_ref[...] = _bdot(xb, wn[...]) + bn_[...]

    nn = bn.shape[0]
    return pl.pallas_call(
        body,
        grid=(_N // _RB,),
        in_specs=[
            pl.BlockSpec((_RB, _D), lambda i: (i, 0)),
            pl.BlockSpec((_D, _D), lambda i: (0, 0)),
            pl.BlockSpec((1, _D), lambda i: (0, 0)),
            pl.BlockSpec((_D, _D), lambda i: (0, 0)),
            pl.BlockSpec((1, _D), lambda i: (0, 0)),
            pl.BlockSpec((_D, 1), lambda i: (0, 0)),
            pl.BlockSpec((1, 1), lambda i: (0, 0)),
            pl.BlockSpec((_D, nn), lambda i: (0, 0)),
            pl.BlockSpec((1, nn), lambda i: (0, 0)),
        ],
        out_specs=[
            pl.BlockSpec((_RB, 1), lambda i: (i, 0)),
            pl.BlockSpec((_RB, nn), lambda i: (i, 0)),
        ],
        out_shape=[
            jax.ShapeDtypeStruct((_N, 1), jnp.float32),
            jax.ShapeDtypeStruct((_N, nn), jnp.float32),
        ],
    )(x3, Wa, ba.reshape(1, _D), Wb, bb.reshape(1, _D), Wc,
      bc.reshape(1, 1), Wn, bn.reshape(1, nn))


def _head(A_t, x3, Wr, br, Wcls, bcls):
    nc = bcls.shape[0]

    def body(a_ref, x_ref, wr, br_, wcls, bcls_, lg_ref, pr_ref, yh_ref,
             s_ref):
        A = a_ref[...]
        m = jnp.max(A, axis=1, keepdims=True)
        e = jnp.exp(A - m)
        p = e / jnp.sum(e, axis=1, keepdims=True)
        hp = _bdot(p, x_ref[...])
        hr = jnp.maximum(_bdot(hp, wr[...]) + br_[...], 0.0)
        lg = _bdot(hr, wcls[...]) + bcls_[...]
        lg_ref[...] = lg
        lm = jnp.max(lg, axis=1, keepdims=True)
        el = jnp.exp(lg - lm)
        pr_ref[...] = el / jnp.sum(el, axis=1, keepdims=True)
        col = lax.broadcasted_iota(jnp.int32, (1, nc), 1)
        yh_ref[...] = jnp.min(jnp.where(lg == lm, col, nc), axis=1,
                              keepdims=True)
        haz = jax.nn.sigmoid(lg)
        lgp = jnp.log(1.0 - haz)
        parts = [lgp[:, 0:1]]
        for k in range(1, nc):
            parts.append(parts[-1] + lgp[:, k:k + 1])
        s_ref[...] = jnp.exp(jnp.concatenate(parts, axis=1))

    return pl.pallas_call(
        body,
        out_shape=[
            jax.ShapeDtypeStruct((1, nc), jnp.float32),
            jax.ShapeDtypeStruct((1, nc), jnp.float32),
            jax.ShapeDtypeStruct((1, 1), jnp.int32),
            jax.ShapeDtypeStruct((1, nc), jnp.float32),
        ],
    )(A_t, x3, Wr, br.reshape(1, _D), Wcls, bcls.reshape(1, nc))


def kernel(x, edge_index, batch, W11, b11, W12, b12, W21, b21, W22, b22,
           W31, b31, W32, b32, Wa, ba, Wb, bb, Wc, bc, Wr, br, Wcls, bcls,
           Wn, bn):
    perm = jnp.argsort(edge_index[1], stable=True)
    sp = edge_index[0][perm]
    dp = edge_index[1][perm]
    nxt = jnp.concatenate([dp[1:], jnp.full((1,), -1, jnp.int32)])
    last = (dp != nxt).reshape(_NS, _EPS)
    last = last.at[:, -1].set(True)
    dpr = dp.reshape(_NS, _EPS)
    idxa = jnp.where(last, dpr, _N)
    pck = ((idxa << 14) | sp.reshape(_NS, _EPS)).reshape(_NS, _CH, _C)
    z = jnp.zeros((320, _HD), jnp.float32)

    def agg(xe, xo):
        return _sc_agg(xe, xo, pck, z)

    xe0, xo0 = x[:, :_HD], x[:, _HD:]
    xe1, xo1 = _mlp(xe0, xo0, agg(xe0, xo0), W11, b11, W12, b12)
    xe2, xo2 = _mlp(xe1, xo1, agg(xe1, xo1), W21, b21, W22, b22)
    xe3, xo3 = _mlp(xe2, xo2, agg(xe2, xo2), W31, b31, W32, b32)
    x3 = jnp.concatenate([xe3, xo3], axis=1)
    A, Y_node = _attn(x3, Wa, ba, Wb, bb, Wc, bc, Wn, bn)
    A_raw = A.reshape(1, _N)
    logits, Y_prob, Y_hat, S = _head(A_raw, x3, Wr, br, Wcls, bcls)
    return (logits, Y_prob, Y_hat, A_raw, S, Y_node)
